# baseline (speedup 1.0000x reference)
import functools

import jax
import jax.numpy as jnp
from jax import lax
from jax.experimental import pallas as pl
from jax.experimental.pallas import tpu as pltpu
from jax.experimental.pallas import tpu_sc as plsc

_DIM = 4096
_EXPERTS = 64
_TOPK = 8
_BT = 512
_LOWMASK = 63
_HIMASK = ~63
_NEG_INF = float("-inf")

_NC = 2
_NS = 16
_LANES = 16


def _router_body(x_ref, w_ref, ts_ref, ti_ref):
    x = x_ref[...]
    w = w_ref[...]
    s = lax.dot_general(
        x, w, (((1,), (1,)), ((), ())), preferred_element_type=jnp.float32
    )
    iota = lax.broadcasted_iota(jnp.int32, s.shape, 1)
    rev = (_EXPERTS - 1) - iota
    bits = lax.bitcast_convert_type(s, jnp.int32)
    emb = jnp.where(bits >= 0, rev, iota)
    key = lax.bitcast_convert_type((bits & _HIMASK) | emb, jnp.float32)

    tops = []
    idxs = []
    for _ in range(_TOPK):
        m = jnp.max(key, axis=1, keepdims=True)
        key = jnp.where(key == m, _NEG_INF, key)
        mbits = lax.bitcast_convert_type(m, jnp.int32)
        low = mbits & _LOWMASK
        idxs.append(jnp.where(mbits >= 0, (_EXPERTS - 1) - low, low))
        tops.append(lax.bitcast_convert_type(mbits & _HIMASK, jnp.float32))
    ts_ref[...] = jax.nn.sigmoid(jnp.concatenate(tops, axis=1))
    ti_ref[...] = jnp.concatenate(idxs, axis=1)


def _tc_router(x, W, base_tok=0, ntok=None):
    ntok = x.shape[0] if ntok is None else ntok
    grid = ntok // _BT
    base_blk = base_tok // _BT
    return pl.pallas_call(
        _router_body,
        grid=(grid,),
        in_specs=[
            pl.BlockSpec((_BT, _DIM), lambda i: (base_blk + i, 0)),
            pl.BlockSpec((_EXPERTS, _DIM), lambda i: (0, 0)),
        ],
        out_specs=[
            pl.BlockSpec((_BT, _TOPK), lambda i: (i, 0)),
            pl.BlockSpec((_BT, _TOPK), lambda i: (i, 0)),
        ],
        out_shape=[
            jax.ShapeDtypeStruct((ntok, _TOPK), jnp.float32),
            jax.ShapeDtypeStruct((ntok, _TOPK), jnp.int32),
        ],
        compiler_params=pltpu.CompilerParams(
            dimension_semantics=("arbitrary",),
        ),
    )(x, W)


def _make_sc_hist(total_idx):
    chunk = total_idx // (_NC * _NS)
    steps = chunk // _LANES
    mesh = plsc.VectorSubcoreMesh(core_axis_name="c", subcore_axis_name="s")

    @functools.partial(
        pl.kernel,
        out_type=jax.ShapeDtypeStruct((_NC, 4, _LANES), jnp.float32),
        mesh=mesh,
        compiler_params=pltpu.CompilerParams(needs_layout_passes=False),
        scratch_types=[
            pltpu.VMEM((chunk,), jnp.int32),
            pltpu.VMEM((_EXPERTS * _LANES,), jnp.float32),
            pltpu.VMEM((4, _LANES), jnp.float32),
            pltpu.VMEM_SHARED((_NS, 4, _LANES), jnp.float32),
        ],
    )
    def hist_kernel(ti_hbm, out_hbm, idx_v, hist_v, part_v, shared_v):
        cid = lax.axis_index("c")
        sid = lax.axis_index("s")
        wid = cid * _NS + sid
        base = wid * chunk
        pltpu.sync_copy(ti_hbm.at[pl.ds(base, chunk)], idx_v)

        zeros16 = jnp.zeros((_LANES,), jnp.float32)
        for r in range(_EXPERTS):
            hist_v[pl.ds(r * _LANES, _LANES)] = zeros16

        lane = lax.broadcasted_iota(jnp.int32, (_LANES,), 0)
        ones16 = jnp.ones((_LANES,), jnp.float32)

        def body(i, carry):
            e = idx_v[pl.ds(i * _LANES, _LANES)]
            slot = lane * _EXPERTS + ((e >> 4) << 4) + (e & 15)
            plsc.addupdate_scatter(hist_v, [slot], ones16)
            return carry

        lax.fori_loop(0, steps, body, 0)

        for h in range(4):
            acc = hist_v[pl.ds(h * _LANES, _LANES)]
            for l in range(1, _LANES):
                acc = acc + hist_v[pl.ds((l * 4 + h) * _LANES, _LANES)]
            part_v[h, :] = acc

        pltpu.sync_copy(part_v, shared_v.at[sid])
        plsc.subcore_barrier()

        @pl.when(sid == 0)
        def _fold_core():
            accs = [zeros16] * 4
            for t in range(_NS):
                pltpu.sync_copy(shared_v.at[t], part_v)
                for h in range(4):
                    accs[h] = accs[h] + part_v[h, :]
            for h in range(4):
                part_v[h, :] = accs[h]
            pltpu.sync_copy(part_v, out_hbm.at[cid])

    return hist_kernel


def kernel(x, W):
    ts, ti = _tc_router(x, W)
    hist = _make_sc_hist(ti.size)
    parts = hist(ti.reshape(-1))
    cnt = parts.sum(axis=0).reshape(_EXPERTS)
    return ts, ti, cnt


# speedup vs baseline: 1.0813x; 1.0813x over previous
import functools

import jax
import jax.numpy as jnp
from jax import lax
from jax.experimental import pallas as pl
from jax.experimental.pallas import tpu as pltpu
from jax.experimental.pallas import tpu_sc as plsc

_DIM = 4096
_EXPERTS = 64
_TOPK = 8
_BT = 1024
_LOWMASK = 63
_HIMASK = ~63
_NEG_INF = float("-inf")

_NC = 2
_NS = 16
_LANES = 16


def _router_body(x_ref, w_ref, ts_ref, ti_ref):
    x = x_ref[...]
    w = w_ref[...]
    s = lax.dot_general(
        x, w, (((1,), (1,)), ((), ())), preferred_element_type=jnp.float32
    )
    iota = lax.broadcasted_iota(jnp.int32, s.shape, 1)
    rev = (_EXPERTS - 1) - iota
    bits = lax.bitcast_convert_type(s, jnp.int32)
    emb = jnp.where(bits >= 0, rev, iota)
    key = lax.bitcast_convert_type((bits & _HIMASK) | emb, jnp.float32)

    tops = []
    idxs = []
    for _ in range(_TOPK):
        m = jnp.max(key, axis=1, keepdims=True)
        key = jnp.where(key == m, _NEG_INF, key)
        mbits = lax.bitcast_convert_type(m, jnp.int32)
        low = mbits & _LOWMASK
        idxs.append(jnp.where(mbits >= 0, (_EXPERTS - 1) - low, low))
        tops.append(lax.bitcast_convert_type(mbits & _HIMASK, jnp.float32))
    ts_ref[...] = jax.nn.sigmoid(jnp.concatenate(tops, axis=1))
    ti_ref[...] = jnp.concatenate(idxs, axis=1)


def _tc_router(x, W, base_tok=0, ntok=None):
    ntok = x.shape[0] if ntok is None else ntok
    grid = ntok // _BT
    base_blk = base_tok // _BT
    return pl.pallas_call(
        _router_body,
        grid=(grid,),
        in_specs=[
            pl.BlockSpec((_BT, _DIM), lambda i: (base_blk + i, 0)),
            pl.BlockSpec((_EXPERTS, _DIM), lambda i: (0, 0)),
        ],
        out_specs=[
            pl.BlockSpec((_BT, _TOPK), lambda i: (i, 0)),
            pl.BlockSpec((_BT, _TOPK), lambda i: (i, 0)),
        ],
        out_shape=[
            jax.ShapeDtypeStruct((ntok, _TOPK), jnp.float32),
            jax.ShapeDtypeStruct((ntok, _TOPK), jnp.int32),
        ],
        compiler_params=pltpu.CompilerParams(
            dimension_semantics=("arbitrary",),
        ),
    )(x, W)


def _make_sc_hist(total_idx):
    chunk = total_idx // _NS
    steps = chunk // _LANES
    mesh = plsc.VectorSubcoreMesh(
        core_axis_name="c", subcore_axis_name="s", num_cores=1
    )

    @functools.partial(
        pl.kernel,
        out_type=jax.ShapeDtypeStruct((4, _LANES), jnp.float32),
        mesh=mesh,
        compiler_params=pltpu.CompilerParams(needs_layout_passes=False),
        scratch_types=[
            pltpu.VMEM((chunk,), jnp.int32),
            pltpu.VMEM((_EXPERTS * _LANES,), jnp.float32),
            pltpu.VMEM((4, _LANES), jnp.float32),
            pltpu.VMEM_SHARED((_NS, 4, _LANES), jnp.float32),
        ],
    )
    def hist_kernel(ti_hbm, out_hbm, idx_v, hist_v, part_v, shared_v):
        sid = lax.axis_index("s")
        base = sid * chunk
        pltpu.sync_copy(ti_hbm.at[pl.ds(base, chunk)], idx_v)

        zeros16 = jnp.zeros((_LANES,), jnp.float32)
        for r in range(_EXPERTS):
            hist_v[pl.ds(r * _LANES, _LANES)] = zeros16

        lane = lax.broadcasted_iota(jnp.int32, (_LANES,), 0)
        ones16 = jnp.ones((_LANES,), jnp.float32)

        def body(i, carry):
            e = idx_v[pl.ds(i * _LANES, _LANES)]
            slot = lane * _EXPERTS + ((e >> 4) << 4) + (e & 15)
            plsc.addupdate_scatter(hist_v, [slot], ones16)
            return carry

        lax.fori_loop(0, steps, body, 0)

        for h in range(4):
            acc = hist_v[pl.ds(h * _LANES, _LANES)]
            for l in range(1, _LANES):
                acc = acc + hist_v[pl.ds((l * 4 + h) * _LANES, _LANES)]
            part_v[h, :] = acc

        pltpu.sync_copy(part_v, shared_v.at[sid])
        plsc.subcore_barrier()

        @pl.when(sid == 0)
        def _fold_core():
            accs = [zeros16] * 4
            for t in range(_NS):
                pltpu.sync_copy(shared_v.at[t], part_v)
                for h in range(4):
                    accs[h] = accs[h] + part_v[h, :]
            for h in range(4):
                part_v[h, :] = accs[h]
            pltpu.sync_copy(part_v, out_hbm)

    return hist_kernel


def kernel(x, W):
    ts, ti = _tc_router(x, W)
    hist = _make_sc_hist(ti.size)
    parts = hist(ti.reshape(-1))
    cnt = parts.reshape(_EXPERTS)
    return ts, ti, cnt


# speedup vs baseline: 1.0823x; 1.0009x over previous
import functools

import jax
import jax.numpy as jnp
from jax import lax
from jax.experimental import pallas as pl
from jax.experimental.pallas import tpu as pltpu
from jax.experimental.pallas import tpu_sc as plsc

_DIM = 4096
_EXPERTS = 64
_TOPK = 8
_BT = 1024
_LOWMASK = 63
_HIMASK = ~63
_NEG_INF = float("-inf")

_NC = 2
_NS = 16
_LANES = 16


def _router_body(x_ref, w_ref, ts_ref, ti_ref):
    x = x_ref[...]
    w = w_ref[...]
    s = lax.dot_general(
        x, w, (((1,), (1,)), ((), ())), preferred_element_type=jnp.float32
    )
    iota = lax.broadcasted_iota(jnp.int32, s.shape, 1)
    rev = (_EXPERTS - 1) - iota
    bits = lax.bitcast_convert_type(s, jnp.int32)
    emb = jnp.where(bits >= 0, rev, iota)
    key = lax.bitcast_convert_type((bits & _HIMASK) | emb, jnp.float32)

    tops = []
    idxs = []
    for _ in range(_TOPK):
        m = jnp.max(key, axis=1, keepdims=True)
        key = jnp.where(key == m, _NEG_INF, key)
        mbits = lax.bitcast_convert_type(m, jnp.int32)
        low = mbits & _LOWMASK
        idxs.append(jnp.where(mbits >= 0, (_EXPERTS - 1) - low, low))
        tops.append(lax.bitcast_convert_type(mbits & _HIMASK, jnp.float32))
    ts_ref[...] = jax.nn.sigmoid(jnp.concatenate(tops, axis=1))
    ti_ref[...] = jnp.concatenate(idxs, axis=1)


def _tc_router(x, W, base_tok=0, ntok=None):
    ntok = x.shape[0] if ntok is None else ntok
    grid = ntok // _BT
    base_blk = base_tok // _BT
    return pl.pallas_call(
        _router_body,
        grid=(grid,),
        in_specs=[
            pl.BlockSpec((_BT, _DIM), lambda i: (base_blk + i, 0)),
            pl.BlockSpec((_EXPERTS, _DIM), lambda i: (0, 0)),
        ],
        out_specs=[
            pl.BlockSpec((_BT, _TOPK), lambda i: (i, 0)),
            pl.BlockSpec((_BT, _TOPK), lambda i: (i, 0)),
        ],
        out_shape=[
            jax.ShapeDtypeStruct((ntok, _TOPK), jnp.float32),
            jax.ShapeDtypeStruct((ntok, _TOPK), jnp.int32),
        ],
        compiler_params=pltpu.CompilerParams(
            dimension_semantics=("arbitrary",),
        ),
    )(x, W)


def _make_sc_hist(total_idx):
    chunk = total_idx // _NS
    steps = chunk // _LANES
    mesh = plsc.VectorSubcoreMesh(
        core_axis_name="c", subcore_axis_name="s", num_cores=1
    )

    @functools.partial(
        pl.kernel,
        out_type=jax.ShapeDtypeStruct((4, _LANES), jnp.float32),
        mesh=mesh,
        compiler_params=pltpu.CompilerParams(
            needs_layout_passes=False,
            disable_bounds_checks=True,
            disable_semaphore_checks=True,
            skip_device_barrier=True,
        ),
        scratch_types=[
            pltpu.VMEM((chunk,), jnp.int32),
            pltpu.VMEM((_EXPERTS * _LANES,), jnp.float32),
            pltpu.VMEM((4, _LANES), jnp.float32),
            pltpu.VMEM_SHARED((_NS, 4, _LANES), jnp.float32),
        ],
    )
    def hist_kernel(ti_hbm, out_hbm, idx_v, hist_v, part_v, shared_v):
        sid = lax.axis_index("s")
        base = sid * chunk
        pltpu.sync_copy(ti_hbm.at[pl.ds(base, chunk)], idx_v)

        zeros16 = jnp.zeros((_LANES,), jnp.float32)
        for r in range(_EXPERTS):
            hist_v[pl.ds(r * _LANES, _LANES)] = zeros16

        lane = lax.broadcasted_iota(jnp.int32, (_LANES,), 0)
        ones16 = jnp.ones((_LANES,), jnp.float32)

        def body(i, carry):
            e = idx_v[pl.ds(i * _LANES, _LANES)]
            slot = lane * _EXPERTS + ((e >> 4) << 4) + (e & 15)
            plsc.addupdate_scatter(hist_v, [slot], ones16)
            return carry

        lax.fori_loop(0, steps, body, 0)

        for h in range(4):
            acc = hist_v[pl.ds(h * _LANES, _LANES)]
            for l in range(1, _LANES):
                acc = acc + hist_v[pl.ds((l * 4 + h) * _LANES, _LANES)]
            part_v[h, :] = acc

        pltpu.sync_copy(part_v, shared_v.at[sid])
        plsc.subcore_barrier()

        @pl.when(sid == 0)
        def _fold_core():
            accs = [zeros16] * 4
            for t in range(_NS):
                pltpu.sync_copy(shared_v.at[t], part_v)
                for h in range(4):
                    accs[h] = accs[h] + part_v[h, :]
            for h in range(4):
                part_v[h, :] = accs[h]
            pltpu.sync_copy(part_v, out_hbm)

    return hist_kernel


def kernel(x, W):
    ts, ti = _tc_router(x, W)
    hist = _make_sc_hist(ti.size)
    parts = hist(ti.reshape(-1))
    cnt = parts.reshape(_EXPERTS)
    return ts, ti, cnt
